# baseline (speedup 1.0000x reference)
import functools

import jax
import jax.numpy as jnp
from jax import lax
from jax.experimental import pallas as pl
from jax.experimental.pallas import tpu as pltpu
from jax.experimental.pallas import tpu_sc as plsc

BATCH = 16384
N_FIELDS = 26
EMBEDDING_DIM = 64

_B = BATCH * N_FIELDS
_NC = 2
_NS = 16
_NW = _NC * _NS
_B_PER_W = _B // _NW
_CHUNK = 128
_N_CHUNKS = _B_PER_W // _CHUNK

_mesh = plsc.VectorSubcoreMesh(core_axis_name="c", subcore_axis_name="s")


@functools.partial(
    pl.kernel,
    mesh=_mesh,
    out_type=jax.ShapeDtypeStruct((_B, EMBEDDING_DIM), jnp.float32),
    scratch_types=[
        pltpu.VMEM((_B_PER_W,), jnp.int32),
        pltpu.VMEM((_CHUNK, EMBEDDING_DIM), jnp.float32),
        pltpu.SemaphoreType.DMA,
    ],
    compiler_params=pltpu.CompilerParams(use_tc_tiling_on_sc=False),
)
def _sc_gather(idx_hbm, table_hbm, out_hbm, idx_v, rows_v, sem):
    wid = lax.axis_index("s") * _NC + lax.axis_index("c")
    base = wid * _B_PER_W
    pltpu.sync_copy(idx_hbm.at[pl.ds(base, _B_PER_W)], idx_v)

    def chunk_body(i, carry):
        off = i * _CHUNK
        pltpu.async_copy(
            table_hbm.at[idx_v.at[pl.ds(off, _CHUNK)]], rows_v, sem
        ).wait()
        pltpu.sync_copy(rows_v, out_hbm.at[pl.ds(base + off, _CHUNK)])
        return carry

    lax.fori_loop(0, _N_CHUNKS, chunk_body, 0)


def kernel(token_ids, weight):
    idx_flat = jnp.reshape(token_ids, (_B,)).astype(jnp.int32)
    out = _sc_gather(idx_flat, weight)
    return jnp.reshape(out, (BATCH, N_FIELDS, EMBEDDING_DIM))


# speedup vs baseline: 1.0741x; 1.0741x over previous
import functools

import jax
import jax.numpy as jnp
from jax import lax
from jax.experimental import pallas as pl
from jax.experimental.pallas import tpu as pltpu
from jax.experimental.pallas import tpu_sc as plsc

BATCH = 16384
N_FIELDS = 26
EMBEDDING_DIM = 64

_B = BATCH * N_FIELDS
_NC = 2
_NS = 16
_NW = _NC * _NS
_B_PER_W = _B // _NW
_CHUNK = 128
_N_CHUNKS = _B_PER_W // _CHUNK
_NBUF = 8
_NGROUPS = _N_CHUNKS // _NBUF

_mesh = plsc.VectorSubcoreMesh(core_axis_name="c", subcore_axis_name="s")


@functools.partial(
    pl.kernel,
    mesh=_mesh,
    out_type=jax.ShapeDtypeStruct((_B, EMBEDDING_DIM), jnp.float32),
    scratch_types=[
        pltpu.VMEM((_B_PER_W,), jnp.int32),
        pltpu.VMEM((_NBUF, _CHUNK, EMBEDDING_DIM), jnp.float32),
        pltpu.SemaphoreType.DMA((_NBUF,)),
        pltpu.SemaphoreType.DMA((_NBUF,)),
    ],
    compiler_params=pltpu.CompilerParams(use_tc_tiling_on_sc=False),
)
def _sc_gather(idx_hbm, table_hbm, out_hbm, idx_v, rows_v, gsems, ssems):
    wid = lax.axis_index("s") * _NC + lax.axis_index("c")
    base = wid * _B_PER_W
    pltpu.sync_copy(idx_hbm.at[pl.ds(base, _B_PER_W)], idx_v)

    def start_gather(chunk, b):
        pltpu.async_copy(
            table_hbm.at[idx_v.at[pl.ds(chunk * _CHUNK, _CHUNK)]],
            rows_v.at[b],
            gsems.at[b],
        )

    def wait_gather(b):
        pltpu.make_async_copy(
            table_hbm.at[idx_v.at[pl.ds(0, _CHUNK)]], rows_v.at[b], gsems.at[b]
        ).wait()

    def start_store(chunk, b):
        pltpu.async_copy(
            rows_v.at[b], out_hbm.at[pl.ds(base + chunk * _CHUNK, _CHUNK)],
            ssems.at[b],
        )

    def wait_store(b):
        pltpu.make_async_copy(
            rows_v.at[b], out_hbm.at[pl.ds(base, _CHUNK)], ssems.at[b]
        ).wait()

    for b in range(_NBUF):
        start_gather(b, b)

    def group_body(g, carry):
        for b in range(_NBUF):
            wait_gather(b)
            start_store(g * _NBUF + b, b)
        for b in range(_NBUF):

            @pl.when(g + 1 < _NGROUPS)
            def _():
                wait_store(b)
                start_gather((g + 1) * _NBUF + b, b)

        return carry

    lax.fori_loop(0, _NGROUPS, group_body, 0)

    for b in range(_NBUF):
        wait_store(b)


def kernel(token_ids, weight):
    idx_flat = jnp.reshape(token_ids, (_B,)).astype(jnp.int32)
    out = _sc_gather(idx_flat, weight)
    return jnp.reshape(out, (BATCH, N_FIELDS, EMBEDDING_DIM))
